# baseline (speedup 1.0000x reference)
import functools

import jax
import jax.numpy as jnp
from jax import lax
from jax.experimental import pallas as pl
from jax.experimental.pallas import tpu as pltpu
from jax.experimental.pallas import tpu_sc as plsc

N_NODES_K = 10000
N_GRAPHS_K = 100
FEAT = 128
H = 10240
ROWS_PER_TILE = H // 16
CHUNK = 128
NW = 32
EPAD = 327680
NCH = EPAD // (NW * CHUNK)
NPAIR = NCH // 2
HCH = NCH // 2
HPAIR = HCH // 2
DEG_W = 128

_mesh = plsc.VectorSubcoreMesh(core_axis_name="c", subcore_axis_name="s")


def _zero_vmem(ref, nrows, ncols):
    z = jnp.zeros((16,), jnp.float32)

    def body(i, carry):
        for j in range(ncols // 16):
            ref[i, pl.ds(j * 16, 16)] = z
        return carry

    lax.fori_loop(0, nrows, body, 0)


def _fill_vmem(ref, nrows, ncols, value):
    v = jnp.full((16,), value, jnp.float32)

    def body(i, carry):
        for j in range(ncols // 16):
            ref[i, pl.ds(j * 16, 16)] = v
        return carry

    lax.fori_loop(0, nrows, body, 0)


@functools.partial(
    pl.kernel,
    mesh=_mesh,
    out_type=jax.ShapeDtypeStruct((2, H, DEG_W), jnp.float32),
    scratch_types=[
        pltpu.VMEM((NCH, CHUNK), jnp.int32),
        pltpu.VMEM((CHUNK, DEG_W), jnp.float32),
        pltpu.VMEM_SHARED((H, DEG_W), jnp.float32),
    ],
)
def _degree_kernel(dst_hbm, out_hbm, dst_v, ones_v, acc_sh):
    c = lax.axis_index("c")
    s = lax.axis_index("s")
    w = c * 16 + s
    row0 = s * ROWS_PER_TILE

    _zero_vmem(ones_v, CHUNK, DEG_W)
    for k in range(ROWS_PER_TILE // CHUNK):
        pltpu.sync_copy(ones_v, acc_sh.at[pl.ds(row0 + k * CHUNK, CHUNK)])
    _fill_vmem(ones_v, CHUNK, DEG_W, 1.0)
    plsc.subcore_barrier()

    pltpu.sync_copy(dst_hbm.at[w], dst_v)

    def chunk(j, carry):
        pltpu.sync_copy(ones_v, acc_sh.at[dst_v.at[j]], add=True)
        return carry

    lax.fori_loop(0, NCH, chunk, 0)
    plsc.subcore_barrier()

    for k in range(ROWS_PER_TILE // CHUNK):
        pltpu.sync_copy(acc_sh.at[pl.ds(row0 + k * CHUNK, CHUNK)], ones_v)
        pltpu.sync_copy(ones_v, out_hbm.at[c, pl.ds(row0 + k * CHUNK, CHUNK)])


@functools.partial(
    pl.kernel,
    mesh=_mesh,
    out_type=jax.ShapeDtypeStruct((2, H, FEAT), jnp.float32),
    scratch_types=[
        pltpu.VMEM((HCH, CHUNK), jnp.int32),
        pltpu.VMEM((HCH, CHUNK), jnp.int32),
        pltpu.VMEM((CHUNK, FEAT), jnp.float32),
        pltpu.VMEM_SHARED((H, FEAT), jnp.float32),
        pltpu.SemaphoreType.DMA,
    ],
)
def _edge_agg_kernel(table_hbm, src_hbm, dst_hbm, out_hbm,
                     src_v, dst_v, rows_a, acc_sh, gsem_a):
    c = lax.axis_index("c")
    s = lax.axis_index("s")
    w = c * 16 + s
    row0 = s * ROWS_PER_TILE

    def gather(j, buf, sem):
        return pltpu.async_copy(table_hbm.at[src_v.at[j]], buf, sem)

    _zero_vmem(rows_a, CHUNK, FEAT)
    for k in range(ROWS_PER_TILE // CHUNK):
        pltpu.sync_copy(rows_a, acc_sh.at[pl.ds(row0 + k * CHUNK, CHUNK)])
    plsc.subcore_barrier()

    for h in range(NCH // HCH):
        pltpu.sync_copy(src_hbm.at[w, pl.ds(h * HCH, HCH)], src_v)
        pltpu.sync_copy(dst_hbm.at[w, pl.ds(h * HCH, HCH)], dst_v)

        def chunk(j, carry):
            gather(j, rows_a, gsem_a).wait()
            pltpu.sync_copy(rows_a, acc_sh.at[dst_v.at[j]], add=True)
            return carry

        lax.fori_loop(0, HCH, chunk, 0)
    plsc.subcore_barrier()

    for k in range(ROWS_PER_TILE // CHUNK):
        pltpu.sync_copy(acc_sh.at[pl.ds(row0 + k * CHUNK, CHUNK)], rows_a)
        pltpu.sync_copy(rows_a, out_hbm.at[c, pl.ds(row0 + k * CHUNK, CHUNK)])


def _deg_dis(degA, degB):
    deg = 1.0 + degA[:, :1] + degB[:, :1]
    return deg, lax.rsqrt(deg)


def _tc_head(a_ref, wdec_ref, bdec_ref, w1_ref, degA_ref, degB_ref,
             table_ref, self_ref):
    x0 = jnp.dot(a_ref[...], wdec_ref[...],
                 preferred_element_type=jnp.float32) + bdec_ref[...]
    y1 = jnp.dot(x0, w1_ref[...], preferred_element_type=jnp.float32)
    deg, dis = _deg_dis(degA_ref, degB_ref)
    table_ref[...] = y1 * dis
    self_ref[...] = y1 / deg


def _tc_mid(aggA_ref, aggB_ref, self_ref, degA_ref, degB_ref, b_ref, w_ref,
            table_ref, self_out_ref):
    deg, dis = _deg_dis(degA_ref, degB_ref)
    x = jnp.maximum(
        (aggA_ref[...] + aggB_ref[...]) * dis + self_ref[...] + b_ref[...], 0.0)
    y = jnp.dot(x, w_ref[...], preferred_element_type=jnp.float32)
    table_ref[...] = y * dis
    self_out_ref[...] = y / deg


def _tc_tail(aggA_ref, aggB_ref, self_ref, degA_ref, degB_ref, b_ref,
             sg_ref, wp1_ref, bp1_ref, wp2_ref, bp2_ref,
             out_ref, pooled_ref):
    deg, dis = _deg_dis(degA_ref, degB_ref)
    x2 = jnp.maximum(
        (aggA_ref[...] + aggB_ref[...]) * dis + self_ref[...] + b_ref[...], 0.0)
    sg = sg_ref[...]

    def body(g, carry):
        m = sg == g
        vals = jnp.where(m, x2, -jnp.inf)
        pooled_ref[pl.ds(g, 1), :] = jnp.max(vals, axis=0, keepdims=True)
        return carry

    lax.fori_loop(0, N_GRAPHS_K, body, 0)

    h = jnp.maximum(
        jnp.dot(pooled_ref[...], wp1_ref[...],
                preferred_element_type=jnp.float32) + bp1_ref[...], 0.0)
    out_ref[...] = jnp.dot(h, wp2_ref[...],
                           preferred_element_type=jnp.float32) + bp2_ref[...]


def _impl(adj, edges, subgraph_idx, W_dec, b_dec, W1, b1, W2, b2,
          Wp1, bp1, Wp2, bp2):
    n_max = adj.shape[-1]
    a_pad = adj.reshape(-1, n_max).astype(jnp.float32)
    a_pad = jnp.pad(a_pad, ((0, H - N_NODES_K), (0, 0)))

    edges32 = edges.astype(jnp.int32)
    src = jnp.pad(edges32[:, 0], (0, EPAD - edges32.shape[0]),
                  constant_values=N_NODES_K)
    dst = jnp.pad(edges32[:, 1], (0, EPAD - edges32.shape[0]),
                  constant_values=N_NODES_K)
    src3 = src.reshape(NW, NCH, CHUNK)
    dst3 = dst.reshape(NW, NCH, CHUNK)

    sg = jnp.pad(subgraph_idx.astype(jnp.int32), (0, H - N_NODES_K),
                 constant_values=N_GRAPHS_K).reshape(H, 1)

    deg_out = _degree_kernel(dst3)
    degA, degB = deg_out[0], deg_out[1]

    b_dec2 = b_dec.reshape(1, FEAT)
    b1_2 = b1.reshape(1, FEAT)
    b2_2 = b2.reshape(1, FEAT)
    bp1_2 = bp1.reshape(1, FEAT)
    bp2_2 = bp2.reshape(1, 1)

    table1, self1 = pl.pallas_call(
        _tc_head,
        out_shape=[
            jax.ShapeDtypeStruct((H, FEAT), jnp.float32),
            jax.ShapeDtypeStruct((H, FEAT), jnp.float32),
        ],
    )(a_pad, W_dec, b_dec2, W1, degA, degB)

    agg1 = _edge_agg_kernel(table1, src3, dst3)

    table2, self2 = pl.pallas_call(
        _tc_mid,
        out_shape=[
            jax.ShapeDtypeStruct((H, FEAT), jnp.float32),
            jax.ShapeDtypeStruct((H, FEAT), jnp.float32),
        ],
    )(agg1[0], agg1[1], self1, degA, degB, b1_2, W2)

    agg2 = _edge_agg_kernel(table2, src3, dst3)

    out = pl.pallas_call(
        _tc_tail,
        out_shape=jax.ShapeDtypeStruct((N_GRAPHS_K, 1), jnp.float32),
        scratch_shapes=[pltpu.VMEM((N_GRAPHS_K, FEAT), jnp.float32)],
    )(agg2[0], agg2[1], self2, degA, degB, b2_2, sg, Wp1, bp1_2, Wp2, bp2_2)

    return out


kernel = jax.jit(_impl)


# speedup vs baseline: 2.1192x; 2.1192x over previous
import functools

import jax
import jax.numpy as jnp
from jax import lax
from jax.experimental import pallas as pl
from jax.experimental.pallas import tpu as pltpu
from jax.experimental.pallas import tpu_sc as plsc

N_NODES_K = 10000
N_GRAPHS_K = 100
FEAT = 128
H = 10240
ROWS_PER_TILE = H // 16
CHUNK = 128
NW = 32
EPAD = 327680
NCH = EPAD // (NW * CHUNK)
NPAIR = NCH // 2
HCH = NCH // 2
HPAIR = HCH // 2
DEG_W = 128

_mesh = plsc.VectorSubcoreMesh(core_axis_name="c", subcore_axis_name="s")


def _zero_vmem(ref, nrows, ncols):
    z = jnp.zeros((16,), jnp.float32)

    def body(i, carry):
        for j in range(ncols // 16):
            ref[i, pl.ds(j * 16, 16)] = z
        return carry

    lax.fori_loop(0, nrows, body, 0)


def _fill_vmem(ref, nrows, ncols, value):
    v = jnp.full((16,), value, jnp.float32)

    def body(i, carry):
        for j in range(ncols // 16):
            ref[i, pl.ds(j * 16, 16)] = v
        return carry

    lax.fori_loop(0, nrows, body, 0)


@functools.partial(
    pl.kernel,
    mesh=_mesh,
    out_type=jax.ShapeDtypeStruct((2, H, DEG_W), jnp.float32),
    scratch_types=[
        pltpu.VMEM((NCH, CHUNK), jnp.int32),
        pltpu.VMEM((CHUNK, DEG_W), jnp.float32),
        pltpu.VMEM_SHARED((H, DEG_W), jnp.float32),
    ],
)
def _degree_kernel(dst_hbm, out_hbm, dst_v, ones_v, acc_sh):
    c = lax.axis_index("c")
    s = lax.axis_index("s")
    w = c * 16 + s
    row0 = s * ROWS_PER_TILE

    _zero_vmem(ones_v, CHUNK, DEG_W)
    for k in range(ROWS_PER_TILE // CHUNK):
        pltpu.sync_copy(ones_v, acc_sh.at[pl.ds(row0 + k * CHUNK, CHUNK)])
    _fill_vmem(ones_v, CHUNK, DEG_W, 1.0)
    plsc.subcore_barrier()

    pltpu.sync_copy(dst_hbm.at[w], dst_v)

    def chunk(j, carry):
        pltpu.sync_copy(ones_v, acc_sh.at[dst_v.at[j]], add=True)
        return carry

    lax.fori_loop(0, NCH, chunk, 0)
    plsc.subcore_barrier()

    for k in range(ROWS_PER_TILE // CHUNK):
        pltpu.sync_copy(acc_sh.at[pl.ds(row0 + k * CHUNK, CHUNK)], ones_v)
        pltpu.sync_copy(ones_v, out_hbm.at[c, pl.ds(row0 + k * CHUNK, CHUNK)])


@functools.partial(
    pl.kernel,
    mesh=_mesh,
    out_type=jax.ShapeDtypeStruct((2, H, FEAT), jnp.float32),
    scratch_types=[
        pltpu.VMEM((HCH, CHUNK), jnp.int32),
        pltpu.VMEM((HCH, CHUNK), jnp.int32),
        pltpu.VMEM((CHUNK, FEAT), jnp.float32),
        pltpu.VMEM_SHARED((H, FEAT), jnp.float32),
        pltpu.SemaphoreType.DMA,
    ],
)
def _edge_agg_kernel(table_hbm, src_hbm, dst_hbm, out_hbm,
                     src_v, dst_v, rows_a, acc_sh, gsem_a):
    c = lax.axis_index("c")
    s = lax.axis_index("s")
    w = c * 16 + s
    row0 = s * ROWS_PER_TILE

    def gather(j, buf, sem):
        return pltpu.async_copy(table_hbm.at[src_v.at[j]], buf, sem)

    _zero_vmem(rows_a, CHUNK, FEAT)
    for k in range(ROWS_PER_TILE // CHUNK):
        pltpu.sync_copy(rows_a, acc_sh.at[pl.ds(row0 + k * CHUNK, CHUNK)])
    plsc.subcore_barrier()

    for h in range(NCH // HCH):
        pltpu.sync_copy(src_hbm.at[w, pl.ds(h * HCH, HCH)], src_v)
        pltpu.sync_copy(dst_hbm.at[w, pl.ds(h * HCH, HCH)], dst_v)

        def chunk(j, carry):
            gather(j, rows_a, gsem_a).wait()
            pltpu.sync_copy(rows_a, acc_sh.at[dst_v.at[j]], add=True)
            return carry

        lax.fori_loop(0, HCH, chunk, 0)
    plsc.subcore_barrier()

    for k in range(ROWS_PER_TILE // CHUNK):
        pltpu.sync_copy(acc_sh.at[pl.ds(row0 + k * CHUNK, CHUNK)], rows_a)
        pltpu.sync_copy(rows_a, out_hbm.at[c, pl.ds(row0 + k * CHUNK, CHUNK)])


def _deg_dis(degA, degB):
    deg = 1.0 + degA[:, :1] + degB[:, :1]
    return deg, lax.rsqrt(deg)


def _tc_head(a_ref, wdec_ref, bdec_ref, w1_ref, degA_ref, degB_ref,
             table_ref, self_ref):
    x0 = jnp.dot(a_ref[...], wdec_ref[...],
                 preferred_element_type=jnp.float32) + bdec_ref[...]
    y1 = jnp.dot(x0, w1_ref[...], preferred_element_type=jnp.float32)
    deg, dis = _deg_dis(degA_ref, degB_ref)
    table_ref[...] = y1 * dis
    self_ref[...] = y1 / deg


def _tc_mid(aggA_ref, aggB_ref, self_ref, degA_ref, degB_ref, b_ref, w_ref,
            table_ref, self_out_ref):
    deg, dis = _deg_dis(degA_ref, degB_ref)
    x = jnp.maximum(
        (aggA_ref[...] + aggB_ref[...]) * dis + self_ref[...] + b_ref[...], 0.0)
    y = jnp.dot(x, w_ref[...], preferred_element_type=jnp.float32)
    table_ref[...] = y * dis
    self_out_ref[...] = y / deg


def _tc_tail(aggA_ref, aggB_ref, self_ref, degA_ref, degB_ref, b_ref,
             sg_ref, wp1_ref, bp1_ref, wp2_ref, bp2_ref,
             out_ref, pooled_ref):
    deg, dis = _deg_dis(degA_ref, degB_ref)
    x2 = jnp.maximum(
        (aggA_ref[...] + aggB_ref[...]) * dis + self_ref[...] + b_ref[...], 0.0)
    sg = sg_ref[...]

    def body(g, carry):
        m = sg == g
        vals = jnp.where(m, x2, -jnp.inf)
        pooled_ref[pl.ds(g, 1), :] = jnp.max(vals, axis=0, keepdims=True)
        return carry

    lax.fori_loop(0, N_GRAPHS_K, body, 0)

    h = jnp.maximum(
        jnp.dot(pooled_ref[...], wp1_ref[...],
                preferred_element_type=jnp.float32) + bp1_ref[...], 0.0)
    out_ref[...] = jnp.dot(h, wp2_ref[...],
                           preferred_element_type=jnp.float32) + bp2_ref[...]


def _impl(adj, edges, subgraph_idx, W_dec, b_dec, W1, b1, W2, b2,
          Wp1, bp1, Wp2, bp2):
    n_max = adj.shape[-1]
    a_pad = adj.reshape(-1, n_max).astype(jnp.float32)
    a_pad = jnp.pad(a_pad, ((0, H - N_NODES_K), (0, 0)))

    edges32 = edges.astype(jnp.int32)
    pad_n = EPAD - edges32.shape[0]
    pad_ids = N_NODES_K + jnp.arange(pad_n, dtype=jnp.int32) % (H - N_NODES_K)
    src = jnp.concatenate([edges32[:, 0], pad_ids])
    dst = jnp.concatenate([edges32[:, 1], pad_ids])
    src3 = src.reshape(NW, NCH, CHUNK)
    dst3 = dst.reshape(NW, NCH, CHUNK)

    sg = jnp.pad(subgraph_idx.astype(jnp.int32), (0, H - N_NODES_K),
                 constant_values=N_GRAPHS_K).reshape(H, 1)

    deg_out = _degree_kernel(dst3)
    degA, degB = deg_out[0], deg_out[1]

    b_dec2 = b_dec.reshape(1, FEAT)
    b1_2 = b1.reshape(1, FEAT)
    b2_2 = b2.reshape(1, FEAT)
    bp1_2 = bp1.reshape(1, FEAT)
    bp2_2 = bp2.reshape(1, 1)

    table1, self1 = pl.pallas_call(
        _tc_head,
        out_shape=[
            jax.ShapeDtypeStruct((H, FEAT), jnp.float32),
            jax.ShapeDtypeStruct((H, FEAT), jnp.float32),
        ],
    )(a_pad, W_dec, b_dec2, W1, degA, degB)

    agg1 = _edge_agg_kernel(table1, src3, dst3)

    table2, self2 = pl.pallas_call(
        _tc_mid,
        out_shape=[
            jax.ShapeDtypeStruct((H, FEAT), jnp.float32),
            jax.ShapeDtypeStruct((H, FEAT), jnp.float32),
        ],
    )(agg1[0], agg1[1], self1, degA, degB, b1_2, W2)

    agg2 = _edge_agg_kernel(table2, src3, dst3)

    out = pl.pallas_call(
        _tc_tail,
        out_shape=jax.ShapeDtypeStruct((N_GRAPHS_K, 1), jnp.float32),
        scratch_shapes=[pltpu.VMEM((N_GRAPHS_K, FEAT), jnp.float32)],
    )(agg2[0], agg2[1], self2, degA, degB, b2_2, sg, Wp1, bp1_2, Wp2, bp2_2)

    return out


kernel = jax.jit(_impl)


# speedup vs baseline: 2.4533x; 1.1577x over previous
import functools

import jax
import jax.numpy as jnp
from jax import lax
from jax.experimental import pallas as pl
from jax.experimental.pallas import tpu as pltpu
from jax.experimental.pallas import tpu_sc as plsc

N_NODES_K = 10000
N_GRAPHS_K = 100
FEAT = 128
H = 10240
ROWS_PER_TILE = H // 16
CHUNK = 128
NW = 32
EPAD = 327680
NCH = EPAD // (NW * CHUNK)
NPAIR = NCH // 2
HCH = NCH // 2
HPAIR = HCH // 2
DEG_W = 128

_mesh = plsc.VectorSubcoreMesh(core_axis_name="c", subcore_axis_name="s")


def _zero_vmem(ref, nrows, ncols):
    z = jnp.zeros((16,), jnp.float32)

    def body(i, carry):
        for j in range(ncols // 16):
            ref[i, pl.ds(j * 16, 16)] = z
        return carry

    lax.fori_loop(0, nrows, body, 0)


def _fill_vmem(ref, nrows, ncols, value):
    v = jnp.full((16,), value, jnp.float32)

    def body(i, carry):
        for j in range(ncols // 16):
            ref[i, pl.ds(j * 16, 16)] = v
        return carry

    lax.fori_loop(0, nrows, body, 0)


@functools.partial(
    pl.kernel,
    mesh=_mesh,
    out_type=jax.ShapeDtypeStruct((2, H, DEG_W), jnp.float32),
    scratch_types=[
        pltpu.VMEM((NCH, CHUNK), jnp.int32),
        pltpu.VMEM((CHUNK, DEG_W), jnp.float32),
        pltpu.VMEM_SHARED((H, DEG_W), jnp.float32),
    ],
)
def _degree_kernel(dst_hbm, out_hbm, dst_v, ones_v, acc_sh):
    c = lax.axis_index("c")
    s = lax.axis_index("s")
    w = c * 16 + s
    row0 = s * ROWS_PER_TILE

    _zero_vmem(ones_v, CHUNK, DEG_W)
    for k in range(ROWS_PER_TILE // CHUNK):
        pltpu.sync_copy(ones_v, acc_sh.at[pl.ds(row0 + k * CHUNK, CHUNK)])
    _fill_vmem(ones_v, CHUNK, DEG_W, 1.0)
    plsc.subcore_barrier()

    pltpu.sync_copy(dst_hbm.at[w], dst_v)

    def chunk(j, carry):
        pltpu.sync_copy(ones_v, acc_sh.at[dst_v.at[j]], add=True)
        return carry

    lax.fori_loop(0, NCH, chunk, 0)
    plsc.subcore_barrier()

    for k in range(ROWS_PER_TILE // CHUNK):
        pltpu.sync_copy(acc_sh.at[pl.ds(row0 + k * CHUNK, CHUNK)], ones_v)
        pltpu.sync_copy(ones_v, out_hbm.at[c, pl.ds(row0 + k * CHUNK, CHUNK)])


@functools.partial(
    pl.kernel,
    mesh=_mesh,
    out_type=jax.ShapeDtypeStruct((2, H, FEAT), jnp.float32),
    scratch_types=[
        pltpu.VMEM((HCH, CHUNK), jnp.int32),
        pltpu.VMEM((HCH, CHUNK), jnp.int32),
        pltpu.VMEM((CHUNK, FEAT), jnp.float32),
        pltpu.VMEM((CHUNK, FEAT), jnp.float32),
        pltpu.VMEM_SHARED((H, FEAT), jnp.float32),
        pltpu.SemaphoreType.DMA,
        pltpu.SemaphoreType.DMA,
    ],
)
def _edge_agg_kernel(table_hbm, src_hbm, dst_hbm, out_hbm,
                     src_v, dst_v, rows_a, rows_b, acc_sh, gsem_a, gsem_b):
    c = lax.axis_index("c")
    s = lax.axis_index("s")
    w = c * 16 + s
    row0 = s * ROWS_PER_TILE

    def gather(j, buf, sem):
        return pltpu.async_copy(table_hbm.at[src_v.at[j]], buf, sem)

    _zero_vmem(rows_a, CHUNK, FEAT)
    for k in range(ROWS_PER_TILE // CHUNK):
        pltpu.sync_copy(rows_a, acc_sh.at[pl.ds(row0 + k * CHUNK, CHUNK)])
    plsc.subcore_barrier()

    for h in range(NCH // HCH):
        pltpu.sync_copy(src_hbm.at[w, pl.ds(h * HCH, HCH)], src_v)
        pltpu.sync_copy(dst_hbm.at[w, pl.ds(h * HCH, HCH)], dst_v)

        gather(0, rows_a, gsem_a)

        def pair(i, carry):
            j0 = 2 * i
            pltpu.make_async_copy(
                table_hbm.at[src_v.at[j0]], rows_a, gsem_a).wait()
            gather(j0 + 1, rows_b, gsem_b)
            pltpu.sync_copy(rows_a, acc_sh.at[dst_v.at[j0]], add=True)
            pltpu.make_async_copy(
                table_hbm.at[src_v.at[j0 + 1]], rows_b, gsem_b).wait()
            jn = lax.select(i + 1 < HPAIR, j0 + 2, 0)
            gather(jn, rows_a, gsem_a)
            pltpu.sync_copy(rows_b, acc_sh.at[dst_v.at[j0 + 1]], add=True)
            return carry

        lax.fori_loop(0, HPAIR, pair, 0)
        pltpu.make_async_copy(table_hbm.at[src_v.at[0]], rows_a, gsem_a).wait()
    plsc.subcore_barrier()

    for k in range(ROWS_PER_TILE // CHUNK):
        pltpu.sync_copy(acc_sh.at[pl.ds(row0 + k * CHUNK, CHUNK)], rows_a)
        pltpu.sync_copy(rows_a, out_hbm.at[c, pl.ds(row0 + k * CHUNK, CHUNK)])


def _deg_dis(degA, degB):
    deg = 1.0 + degA[:, :1] + degB[:, :1]
    return deg, lax.rsqrt(deg)


def _tc_head(a_ref, wdec_ref, bdec_ref, w1_ref, degA_ref, degB_ref,
             table_ref, self_ref):
    x0 = jnp.dot(a_ref[...], wdec_ref[...],
                 preferred_element_type=jnp.float32) + bdec_ref[...]
    y1 = jnp.dot(x0, w1_ref[...], preferred_element_type=jnp.float32)
    deg, dis = _deg_dis(degA_ref, degB_ref)
    table_ref[...] = y1 * dis
    self_ref[...] = y1 / deg


def _tc_mid(aggA_ref, aggB_ref, self_ref, degA_ref, degB_ref, b_ref, w_ref,
            table_ref, self_out_ref):
    deg, dis = _deg_dis(degA_ref, degB_ref)
    x = jnp.maximum(
        (aggA_ref[...] + aggB_ref[...]) * dis + self_ref[...] + b_ref[...], 0.0)
    y = jnp.dot(x, w_ref[...], preferred_element_type=jnp.float32)
    table_ref[...] = y * dis
    self_out_ref[...] = y / deg


def _tc_tail(aggA_ref, aggB_ref, self_ref, degA_ref, degB_ref, b_ref,
             sg_ref, wp1_ref, bp1_ref, wp2_ref, bp2_ref,
             out_ref, pooled_ref):
    deg, dis = _deg_dis(degA_ref, degB_ref)
    x2 = jnp.maximum(
        (aggA_ref[...] + aggB_ref[...]) * dis + self_ref[...] + b_ref[...], 0.0)
    sg = sg_ref[...]

    def body(g, carry):
        m = sg == g
        vals = jnp.where(m, x2, -jnp.inf)
        pooled_ref[pl.ds(g, 1), :] = jnp.max(vals, axis=0, keepdims=True)
        return carry

    lax.fori_loop(0, N_GRAPHS_K, body, 0)

    h = jnp.maximum(
        jnp.dot(pooled_ref[...], wp1_ref[...],
                preferred_element_type=jnp.float32) + bp1_ref[...], 0.0)
    out_ref[...] = jnp.dot(h, wp2_ref[...],
                           preferred_element_type=jnp.float32) + bp2_ref[...]


def _impl(adj, edges, subgraph_idx, W_dec, b_dec, W1, b1, W2, b2,
          Wp1, bp1, Wp2, bp2):
    n_max = adj.shape[-1]
    a_pad = adj.reshape(-1, n_max).astype(jnp.float32)
    a_pad = jnp.pad(a_pad, ((0, H - N_NODES_K), (0, 0)))

    edges32 = edges.astype(jnp.int32)
    pad_n = EPAD - edges32.shape[0]
    pad_ids = N_NODES_K + jnp.arange(pad_n, dtype=jnp.int32) % (H - N_NODES_K)
    src = jnp.concatenate([edges32[:, 0], pad_ids])
    dst = jnp.concatenate([edges32[:, 1], pad_ids])
    src3 = src.reshape(NW, NCH, CHUNK)
    dst3 = dst.reshape(NW, NCH, CHUNK)

    sg = jnp.pad(subgraph_idx.astype(jnp.int32), (0, H - N_NODES_K),
                 constant_values=N_GRAPHS_K).reshape(H, 1)

    deg_out = _degree_kernel(dst3)
    degA, degB = deg_out[0], deg_out[1]

    b_dec2 = b_dec.reshape(1, FEAT)
    b1_2 = b1.reshape(1, FEAT)
    b2_2 = b2.reshape(1, FEAT)
    bp1_2 = bp1.reshape(1, FEAT)
    bp2_2 = bp2.reshape(1, 1)

    table1, self1 = pl.pallas_call(
        _tc_head,
        out_shape=[
            jax.ShapeDtypeStruct((H, FEAT), jnp.float32),
            jax.ShapeDtypeStruct((H, FEAT), jnp.float32),
        ],
    )(a_pad, W_dec, b_dec2, W1, degA, degB)

    agg1 = _edge_agg_kernel(table1, src3, dst3)

    table2, self2 = pl.pallas_call(
        _tc_mid,
        out_shape=[
            jax.ShapeDtypeStruct((H, FEAT), jnp.float32),
            jax.ShapeDtypeStruct((H, FEAT), jnp.float32),
        ],
    )(agg1[0], agg1[1], self1, degA, degB, b1_2, W2)

    agg2 = _edge_agg_kernel(table2, src3, dst3)

    out = pl.pallas_call(
        _tc_tail,
        out_shape=jax.ShapeDtypeStruct((N_GRAPHS_K, 1), jnp.float32),
        scratch_shapes=[pltpu.VMEM((N_GRAPHS_K, FEAT), jnp.float32)],
    )(agg2[0], agg2[1], self2, degA, degB, b2_2, sg, Wp1, bp1_2, Wp2, bp2_2)

    return out


kernel = jax.jit(_impl)
